# baseline (speedup 1.0000x reference)
import functools

import jax
import jax.numpy as jnp
from jax import lax
from jax.experimental import pallas as pl
from jax.experimental.pallas import tpu as pltpu
from jax.experimental.pallas import tpu_sc as plsc

SEQ = 200
BATCH = 16384
NUM_WORDS = 1_000_000
EMB_DIM = 32


def _phase_a_body(x_ref, w_ref, b_ref, q_ref):
    x = x_ref[0]
    s = jnp.dot(x, w_ref[...], preferred_element_type=jnp.float32)
    q_ref[...] = ((s + b_ref[0, 0]) * (1.0 / SEQ))[None]


def _phase_a(emb3, w64, lin_b11):
    return pl.pallas_call(
        _phase_a_body,
        grid=(125,),
        in_specs=[
            pl.BlockSpec((1, 125, 64 * EMB_DIM), lambda i: (i, 0, 0)),
            pl.BlockSpec((64 * EMB_DIM, 64), lambda i: (0, 0)),
            pl.BlockSpec((1, 1), lambda i: (0, 0)),
        ],
        out_specs=pl.BlockSpec((1, 125, 64), lambda i: (i, 0, 0)),
        out_shape=jax.ShapeDtypeStruct((125, 125, 64), jnp.float32),
    )(emb3, w64, lin_b11)


_NW = 32
_COLS_W = BATCH // _NW
_G = 8
_NCH = SEQ // _G


_CHUNK = _G * _COLS_W
_WORDS_W = SEQ * _COLS_W


def _phase_b_body(text_hbm, q_hbm, out_hbm, ibuf0, ibuf1, buf0, buf1, acc,
                  ss0, ss1, sg0, sg1):
    wid = lax.axis_index("s") * 2 + lax.axis_index("c")
    col0 = wid * _COLS_W

    def stage(g, ibuf, sem):
        for r in range(_G):
            pltpu.async_copy(text_hbm.at[g * _G + r, pl.ds(col0, _COLS_W)],
                             ibuf.at[pl.ds(r * _COLS_W, _COLS_W)], sem)

    def sdrain(ibuf, sem):
        for r in range(_G):
            pltpu.make_async_copy(text_hbm.at[0, pl.ds(0, _COLS_W)],
                                  ibuf.at[pl.ds(0, _COLS_W)], sem).wait()

    for t in range(32):
        acc[pl.ds(t * 16, 16)] = jnp.zeros((16,), jnp.float32)

    def accumulate(buf):
        for t in range(32):
            v = buf[pl.ds(t * 16, 16)]
            for sp in range(1, _G):
                v = v + buf[pl.ds(sp * _COLS_W + t * 16, 16)]
            sl = pl.ds(t * 16, 16)
            acc[sl] = acc[sl] + v

    def gather(ibuf, buf, sem):
        pltpu.async_copy(q_hbm.at[ibuf], buf, sem)

    def gwait(buf, sem):
        pltpu.make_async_copy(q_hbm.at[ibuf0], buf, sem).wait()

    stage(0, ibuf0, ss0)
    sdrain(ibuf0, ss0)
    gather(ibuf0, buf0, sg0)
    stage(1, ibuf1, ss1)

    def step(g, _):
        even = g % 2 == 0

        @pl.when(even)
        def _():
            sdrain(ibuf1, ss1)
            gather(ibuf1, buf1, sg1)
            gwait(buf0, sg0)
            accumulate(buf0)

            @pl.when(g + 2 < _NCH)
            def _():
                stage(g + 2, ibuf0, ss0)

        @pl.when(jnp.logical_not(even))
        def _():
            sdrain(ibuf0, ss0)
            gather(ibuf0, buf0, sg0)
            gwait(buf1, sg1)
            accumulate(buf1)

            @pl.when(g + 2 < _NCH)
            def _():
                stage(g + 2, ibuf1, ss1)

        return 0

    lax.fori_loop(0, _NCH - 1, step, 0)
    gwait(buf0, sg0)
    accumulate(buf0)

    for t in range(32):
        sl = pl.ds(t * 16, 16)
        v = acc[sl]
        acc[sl] = 1.0 / (1.0 + jnp.exp(-v))
    pltpu.sync_copy(acc, out_hbm.at[pl.ds(col0, _COLS_W)])


def _run(text2d, q):
    mesh = plsc.VectorSubcoreMesh(core_axis_name="c", subcore_axis_name="s")
    f = pl.kernel(
        _phase_b_body,
        out_type=jax.ShapeDtypeStruct((BATCH,), jnp.float32),
        mesh=mesh,
        scratch_types=[
            pltpu.VMEM((_CHUNK,), jnp.int32),
            pltpu.VMEM((_CHUNK,), jnp.int32),
            pltpu.VMEM((_CHUNK,), jnp.float32),
            pltpu.VMEM((_CHUNK,), jnp.float32),
            pltpu.VMEM((_COLS_W,), jnp.float32),
            pltpu.SemaphoreType.DMA,
            pltpu.SemaphoreType.DMA,
            pltpu.SemaphoreType.DMA,
            pltpu.SemaphoreType.DMA,
        ],
    )
    return f(text2d, q)


def kernel(text, emb_weight, lin_w, lin_b):
    k_idx = jnp.arange(64 * EMB_DIM)
    w64 = jnp.where(
        (k_idx[:, None] // EMB_DIM) == jnp.arange(64)[None, :],
        jnp.tile(lin_w[0], 64)[:, None],
        0.0,
    ).astype(jnp.float32)
    q2 = _phase_a(
        emb_weight.reshape(125, 125, 64 * EMB_DIM),
        w64,
        lin_b.reshape(1, 1),
    )
    q = q2.reshape(NUM_WORDS)
    out = _run(text, q)
    return out.reshape(BATCH, 1)


# speedup vs baseline: 1.4949x; 1.4949x over previous
import functools

import jax
import jax.numpy as jnp
from jax import lax
from jax.experimental import pallas as pl
from jax.experimental.pallas import tpu as pltpu
from jax.experimental.pallas import tpu_sc as plsc

SEQ = 200
BATCH = 16384
NUM_WORDS = 1_000_000
EMB_DIM = 32


def _phase_a_body(x_ref, w_ref, b_ref, q_ref):
    w = w_ref[0, :]
    x = x_ref[...]
    s = jnp.sum(x * w[None, None, None, :], axis=-1)
    q_ref[...] = (s + b_ref[0, 0]) * (1.0 / SEQ)


def _phase_a(emb4, lin_w, lin_b11):
    return pl.pallas_call(
        _phase_a_body,
        grid=(25,),
        in_specs=[
            pl.BlockSpec((1, 625, 64, EMB_DIM), lambda i: (i, 0, 0, 0)),
            pl.BlockSpec((1, EMB_DIM), lambda i: (0, 0)),
            pl.BlockSpec((1, 1), lambda i: (0, 0)),
        ],
        out_specs=pl.BlockSpec((1, 625, 64), lambda i: (i, 0, 0)),
        out_shape=jax.ShapeDtypeStruct((25, 625, 64), jnp.float32),
    )(emb4, lin_w, lin_b11)


_NW = 32
_COLS_W = BATCH // _NW
_G = 8
_NCH = SEQ // _G


_CHUNK = _G * _COLS_W
_WORDS_W = SEQ * _COLS_W


def _phase_b_body(text_hbm, q_hbm, out_hbm, ibuf0, ibuf1, buf0, buf1, acc,
                  ss0, ss1, sg0, sg1):
    wid = lax.axis_index("s") * 2 + lax.axis_index("c")
    col0 = wid * _COLS_W

    def stage(g, ibuf, sem):
        for r in range(_G):
            pltpu.async_copy(text_hbm.at[g * _G + r, pl.ds(col0, _COLS_W)],
                             ibuf.at[pl.ds(r * _COLS_W, _COLS_W)], sem)

    def sdrain(ibuf, sem):
        for r in range(_G):
            pltpu.make_async_copy(text_hbm.at[0, pl.ds(0, _COLS_W)],
                                  ibuf.at[pl.ds(0, _COLS_W)], sem).wait()

    for t in range(32):
        acc[pl.ds(t * 16, 16)] = jnp.zeros((16,), jnp.float32)

    def accumulate(buf):
        for t in range(32):
            v = buf[pl.ds(t * 16, 16)]
            for sp in range(1, _G):
                v = v + buf[pl.ds(sp * _COLS_W + t * 16, 16)]
            sl = pl.ds(t * 16, 16)
            acc[sl] = acc[sl] + v

    def gather(ibuf, buf, sem):
        pltpu.async_copy(q_hbm.at[ibuf], buf, sem)

    def gwait(buf, sem):
        pltpu.make_async_copy(q_hbm.at[ibuf0], buf, sem).wait()

    stage(0, ibuf0, ss0)
    sdrain(ibuf0, ss0)
    gather(ibuf0, buf0, sg0)
    stage(1, ibuf1, ss1)

    def step(g, _):
        even = g % 2 == 0

        @pl.when(even)
        def _():
            sdrain(ibuf1, ss1)
            gather(ibuf1, buf1, sg1)
            gwait(buf0, sg0)
            accumulate(buf0)

            @pl.when(g + 2 < _NCH)
            def _():
                stage(g + 2, ibuf0, ss0)

        @pl.when(jnp.logical_not(even))
        def _():
            sdrain(ibuf0, ss0)
            gather(ibuf0, buf0, sg0)
            gwait(buf1, sg1)
            accumulate(buf1)

            @pl.when(g + 2 < _NCH)
            def _():
                stage(g + 2, ibuf1, ss1)

        return 0

    lax.fori_loop(0, _NCH - 1, step, 0)
    gwait(buf0, sg0)
    accumulate(buf0)

    for t in range(32):
        sl = pl.ds(t * 16, 16)
        v = acc[sl]
        acc[sl] = 1.0 / (1.0 + jnp.exp(-v))
    pltpu.sync_copy(acc, out_hbm.at[pl.ds(col0, _COLS_W)])


def _run(text2d, q):
    mesh = plsc.VectorSubcoreMesh(core_axis_name="c", subcore_axis_name="s")
    f = pl.kernel(
        _phase_b_body,
        out_type=jax.ShapeDtypeStruct((BATCH,), jnp.float32),
        mesh=mesh,
        scratch_types=[
            pltpu.VMEM((_CHUNK,), jnp.int32),
            pltpu.VMEM((_CHUNK,), jnp.int32),
            pltpu.VMEM((_CHUNK,), jnp.float32),
            pltpu.VMEM((_CHUNK,), jnp.float32),
            pltpu.VMEM((_COLS_W,), jnp.float32),
            pltpu.SemaphoreType.DMA,
            pltpu.SemaphoreType.DMA,
            pltpu.SemaphoreType.DMA,
            pltpu.SemaphoreType.DMA,
        ],
    )
    return f(text2d, q)


def kernel(text, emb_weight, lin_w, lin_b):
    q2 = _phase_a(
        emb_weight.reshape(25, 625, 64, EMB_DIM),
        lin_w,
        lin_b.reshape(1, 1),
    )
    q = q2.reshape(NUM_WORDS)
    out = _run(text, q)
    return out.reshape(BATCH, 1)
